# Initial kernel scaffold; baseline (speedup 1.0000x reference)
import jax
import jax.numpy as jnp
from jax.experimental import pallas as pl


def kernel(x, edge_index, W_in, b_in, Wg0, bg0, gamma0, beta0, Wg1, bg1, gamma1, beta1, Wg2, bg2, gamma2, beta2, Wc1, bc1, Wc2, bc2):
    raise NotImplementedError("write your pallas kernel here")


# speedup vs baseline: 1.0185x; 1.0185x over previous
import jax
import jax.numpy as jnp
from jax.experimental import pallas as pl
from jax.experimental.pallas import tpu as pltpu


def _head(h_ref, wc1_ref, bc1_ref, wc2_ref, bc2_ref, out_ref):
    h = h_ref[...]
    t = jnp.maximum(jnp.dot(h, wc1_ref[...], preferred_element_type=jnp.float32) + bc1_ref[...], 0.0)
    out_ref[...] = jnp.dot(t, wc2_ref[...], preferred_element_type=jnp.float32) + bc2_ref[...]


def kernel(x, edge_index, W_in, b_in, Wg0, bg0, gamma0, beta0, Wg1, bg1, gamma1, beta1, Wg2, bg2, gamma2, beta2, Wc1, bc1, Wc2, bc2):
    n = x.shape[0]
    src = edge_index[0]
    dst = edge_index[1]
    loop = jnp.arange(n, dtype=src.dtype)
    s = jnp.concatenate([src, loop])
    d = jnp.concatenate([dst, loop])
    ones = jnp.ones((s.shape[0],), jnp.float32)
    deg = jax.ops.segment_sum(ones, d, num_segments=n)
    dinv = jnp.where(deg > 0, jax.lax.rsqrt(deg), 0.0)
    norm = dinv[s] * dinv[d]

    h = jax.nn.relu(x @ W_in + b_in)
    layers = [(Wg0, bg0, gamma0, beta0), (Wg1, bg1, gamma1, beta1), (Wg2, bg2, gamma2, beta2)]
    for (Wg, bg, g, b) in layers:
        h = h @ Wg
        msg = h[s] * norm[:, None]
        h = jax.ops.segment_sum(msg, d, num_segments=n) + bg
        mu = jnp.mean(h, axis=0)
        var = jnp.mean((h - mu) ** 2, axis=0)
        h = (h - mu) * jax.lax.rsqrt(var + 1e-5) * g + b
        h = jax.nn.relu(h)

    out = pl.pallas_call(
        _head,
        out_shape=jax.ShapeDtypeStruct((n, 1), jnp.float32),
    )(h, Wc1, bc1, Wc2, bc2)
    return out[:, 0]


# speedup vs baseline: 14.7714x; 14.5037x over previous
import functools

import jax
import jax.numpy as jnp
from jax import lax
from jax.experimental import pallas as pl
from jax.experimental.pallas import tpu as pltpu
from jax.experimental.pallas import tpu_sc as plsc

N = 50000
H = 64
HH = 32
E = 800000
EPS = 1e-5

NC = 2
NS = 16
NP = 50176
TROWS = NP // NS
CW = 128
EP = 819200
NCHUNK = EP // CW
CPT = NCHUNK // NS
G = 4
CPW = NCHUNK // (NC * NS)
GD = 8

_mesh = plsc.VectorSubcoreMesh(core_axis_name="c", subcore_axis_name="s")


@functools.partial(
    pl.kernel,
    out_type=jax.ShapeDtypeStruct((NC * NP,), jnp.float32),
    mesh=_mesh,
    compiler_params=pltpu.CompilerParams(use_tc_tiling_on_sc=False),
    scratch_types=[
        pltpu.VMEM_SHARED((NP,), jnp.float32),
        pltpu.VMEM((TROWS,), jnp.float32),
        pltpu.VMEM((CW,), jnp.float32),
        pltpu.VMEM((GD, CW), jnp.int32),
        pltpu.SemaphoreType.DMA,
        pltpu.SemaphoreType.DMA,
    ],
)
def _sc_degree(dst_hbm, out_hbm, acc, zbuf, ones_v, didx, stage_sem, add_sem):
    c = lax.axis_index("c")
    s = lax.axis_index("s")

    def _fill(i, carry):
        zbuf[pl.ds(i * 16, 16)] = jnp.zeros((16,), jnp.float32)
        return carry

    lax.fori_loop(0, TROWS // 16, _fill, 0)
    for i in range(CW // 16):
        ones_v[pl.ds(i * 16, 16)] = jnp.ones((16,), jnp.float32)
    pltpu.sync_copy(zbuf, acc.at[pl.ds(s * TROWS, TROWS)])
    plsc.subcore_barrier()

    wid = c * NS + s

    def _group(k, carry):
        base = wid * CPW + k * GD
        pltpu.async_copy(dst_hbm.at[pl.ds(base, GD)], didx, stage_sem).wait()
        cps = []
        for j in range(GD):
            cps.append(pltpu.async_copy(ones_v, acc.at[didx.at[j]], add_sem, add=True))
        for cp in cps:
            cp.wait()
        return carry

    lax.fori_loop(0, CPW // GD, _group, 0)
    plsc.subcore_barrier()
    pltpu.sync_copy(acc.at[pl.ds(s * TROWS, TROWS)], zbuf)
    pltpu.sync_copy(zbuf, out_hbm.at[pl.ds(c * NP + s * TROWS, TROWS)])


@functools.partial(
    pl.kernel,
    out_type=jax.ShapeDtypeStruct((NC * NP, HH), jnp.float32),
    mesh=_mesh,
    compiler_params=pltpu.CompilerParams(use_tc_tiling_on_sc=False),
    scratch_types=[
        pltpu.VMEM_SHARED((NP, HH), jnp.float32),
        pltpu.VMEM((G, CW), jnp.int32),
        pltpu.VMEM((G, CW), jnp.int32),
        pltpu.VMEM((G, CW, HH), jnp.float32),
        pltpu.SemaphoreType.DMA,
        pltpu.SemaphoreType.DMA,
        pltpu.SemaphoreType.DMA,
    ],
)
def _sc_layer(g2_hbm, src_hbm, dst_hbm, out_hbm, acc, sidx, didx, rows,
              stage_sem, gather_sem, add_sem):
    c = lax.axis_index("c")
    s = lax.axis_index("s")
    off = c * NP

    pltpu.sync_copy(g2_hbm.at[pl.ds(off + s * TROWS, TROWS)],
                    acc.at[pl.ds(s * TROWS, TROWS)])
    plsc.subcore_barrier()

    def _group(k, carry):
        base = s * CPT + k * G
        cp_s = pltpu.async_copy(src_hbm.at[pl.ds(base, G)], sidx, stage_sem)
        cp_d = pltpu.async_copy(dst_hbm.at[pl.ds(base, G)], didx, stage_sem)
        cp_s.wait()
        cp_d.wait()
        for j in range(G):
            for v in range(CW // 16):
                t = sidx[j, pl.ds(v * 16, 16)]
                sidx[j, pl.ds(v * 16, 16)] = t + off
        cps = []
        for j in range(G):
            cps.append(pltpu.async_copy(g2_hbm.at[sidx.at[j]], rows.at[j], gather_sem))
        for cp in cps:
            cp.wait()
        cps = []
        for j in range(G):
            cps.append(pltpu.async_copy(rows.at[j], acc.at[didx.at[j]], add_sem, add=True))
        for cp in cps:
            cp.wait()
        return carry

    lax.fori_loop(0, CPT // G, _group, 0)
    plsc.subcore_barrier()
    pltpu.sync_copy(acc.at[pl.ds(s * TROWS, TROWS)],
                    out_hbm.at[pl.ds(off + s * TROWS, TROWS)])


RB = 6272
NB = NP // RB

_full = lambda *shape: pl.BlockSpec(shape, lambda b: tuple(0 for _ in shape))
_rows = lambda *shape: pl.BlockSpec(shape, lambda b: (b,) + tuple(0 for _ in shape[1:]))
_half = pl.BlockSpec((NC, RB, HH), lambda b: (0, b, 0))


def _tc_front_body(x_ref, w_in_ref, b_in_ref, dv_ref, wg_ref, g2_ref):
    h = jnp.maximum(
        jnp.dot(x_ref[...], w_in_ref[...], preferred_element_type=jnp.float32)
        + b_in_ref[...][None, :], 0.0)
    hw = jnp.dot(h, wg_ref[...], preferred_element_type=jnp.float32)
    g = hw * dv_ref[...]
    g2_ref[0] = g[:, :HH]
    g2_ref[1] = g[:, HH:]


_tc_front = pl.pallas_call(
    _tc_front_body,
    grid=(NB,),
    in_specs=[_rows(RB, 8), _full(8, H), _full(H), _rows(RB, H), _full(H, H)],
    out_specs=_half,
    out_shape=jax.ShapeDtypeStruct((NC, NP, HH), jnp.float32),
)


def _pre_bn(acc2_ref, dv_ref, bg_ref):
    accf = jnp.concatenate([acc2_ref[0], acc2_ref[1]], axis=1)
    return accf * dv_ref[...] + bg_ref[...][None, :]


def _tc_stats_body(acc2_ref, dv_ref, bg_ref, stats_ref):
    b = pl.program_id(0)
    pre = _pre_bn(acc2_ref, dv_ref, bg_ref)
    row = lax.broadcasted_iota(jnp.int32, (RB, H), 0) + b * RB
    prem = jnp.where(row < N, pre, 0.0)

    @pl.when(b == 0)
    def _():
        stats_ref[...] = jnp.zeros((8, H), jnp.float32)

    stats_ref[0, :] += jnp.sum(prem, axis=0)
    stats_ref[1, :] += jnp.sum(prem * prem, axis=0)


_tc_stats = pl.pallas_call(
    _tc_stats_body,
    grid=(NB,),
    in_specs=[_half, _rows(RB, H), _full(H)],
    out_specs=_full(8, H),
    out_shape=jax.ShapeDtypeStruct((8, H), jnp.float32),
)


def _bn_relu(acc2_ref, dv_ref, bg_ref, stats_ref, gamma_ref, beta_ref):
    pre = _pre_bn(acc2_ref, dv_ref, bg_ref)
    mu = stats_ref[0, :] / float(N)
    var = stats_ref[1, :] / float(N) - mu * mu
    h = (pre - mu[None, :]) * lax.rsqrt(var + EPS)[None, :] * gamma_ref[...][None, :] \
        + beta_ref[...][None, :]
    return jnp.maximum(h, 0.0)


def _tc_apply_body(acc2_ref, dv_ref, bg_ref, stats_ref, gamma_ref, beta_ref,
                   wg_ref, g2_ref):
    h = _bn_relu(acc2_ref, dv_ref, bg_ref, stats_ref, gamma_ref, beta_ref)
    hw = jnp.dot(h, wg_ref[...], preferred_element_type=jnp.float32)
    g = hw * dv_ref[...]
    g2_ref[0] = g[:, :HH]
    g2_ref[1] = g[:, HH:]


_tc_apply = pl.pallas_call(
    _tc_apply_body,
    grid=(NB,),
    in_specs=[_half, _rows(RB, H), _full(H), _full(8, H), _full(H), _full(H),
              _full(H, H)],
    out_specs=_half,
    out_shape=jax.ShapeDtypeStruct((NC, NP, HH), jnp.float32),
)


def _tc_head_body(acc2_ref, dv_ref, bg_ref, stats_ref, gamma_ref, beta_ref,
                  wc1_ref, bc1_ref, wc2_ref, bc2_ref, out_ref):
    h = _bn_relu(acc2_ref, dv_ref, bg_ref, stats_ref, gamma_ref, beta_ref)
    t = jnp.maximum(
        jnp.dot(h, wc1_ref[...], preferred_element_type=jnp.float32)
        + bc1_ref[...][None, :], 0.0)
    out_ref[...] = jnp.dot(t, wc2_ref[...], preferred_element_type=jnp.float32) \
        + bc2_ref[...][None, :]


_tc_head = pl.pallas_call(
    _tc_head_body,
    grid=(NB,),
    in_specs=[_half, _rows(RB, H), _full(H), _full(8, H), _full(H), _full(H),
              _full(H, HH), _full(HH), _full(HH, 1), _full(1)],
    out_specs=_rows(RB, 1),
    out_shape=jax.ShapeDtypeStruct((NP, 1), jnp.float32),
)


def kernel(x, edge_index, W_in, b_in, Wg0, bg0, gamma0, beta0, Wg1, bg1, gamma1,
           beta1, Wg2, bg2, gamma2, beta2, Wc1, bc1, Wc2, bc2):
    pad = jnp.full((EP - E,), NP - 1, jnp.int32)
    src2 = jnp.concatenate([edge_index[0], pad]).reshape(NCHUNK, CW)
    dst2 = jnp.concatenate([edge_index[1], pad]).reshape(NCHUNK, CW)
    x8 = jnp.pad(x, ((0, NP - N), (0, 3)))
    w8 = jnp.pad(W_in, ((0, 3), (0, 0)))

    deg_flat = _sc_degree(dst2)
    deg = deg_flat[:N] + deg_flat[NP:NP + N] + 1.0
    dinv64 = jnp.pad(jnp.broadcast_to(lax.rsqrt(deg)[:, None], (N, H)),
                     ((0, NP - N), (0, 0)))

    g2 = _tc_front(x8, w8, b_in, dinv64, Wg0)

    for bg, gamma, beta, wg in ((bg0, gamma0, beta0, Wg1),
                                (bg1, gamma1, beta1, Wg2)):
        acc2 = _sc_layer(g2.reshape(NC * NP, HH), src2, dst2).reshape(NC, NP, HH)
        stats = _tc_stats(acc2, dinv64, bg)
        g2 = _tc_apply(acc2, dinv64, bg, stats, gamma, beta, wg)

    acc2 = _sc_layer(g2.reshape(NC * NP, HH), src2, dst2).reshape(NC, NP, HH)
    stats = _tc_stats(acc2, dinv64, bg2)
    out = _tc_head(acc2, dinv64, bg2, stats, gamma2, beta2, Wc1, bc1, Wc2, bc2)
    return out[:N, 0]


# speedup vs baseline: 17.4453x; 1.1810x over previous
import functools

import jax
import jax.numpy as jnp
from jax import lax
from jax.experimental import pallas as pl
from jax.experimental.pallas import tpu as pltpu
from jax.experimental.pallas import tpu_sc as plsc

N = 50000
H = 64
HH = 32
E = 800000
EPS = 1e-5

NC = 2
NS = 16
NP = 50176
TROWS = NP // NS
CW = 128
EP = 819200
NCHUNK = EP // CW
CPT = NCHUNK // NS
G = 2
NG = CPT // G
CPW = NCHUNK // (NC * NS)
GD = 8

_mesh = plsc.VectorSubcoreMesh(core_axis_name="c", subcore_axis_name="s")


@functools.partial(
    pl.kernel,
    out_type=jax.ShapeDtypeStruct((NC * NP,), jnp.float32),
    mesh=_mesh,
    compiler_params=pltpu.CompilerParams(use_tc_tiling_on_sc=False),
    scratch_types=[
        pltpu.VMEM_SHARED((NP,), jnp.float32),
        pltpu.VMEM((TROWS,), jnp.float32),
        pltpu.VMEM((CW,), jnp.float32),
        pltpu.VMEM((GD, CW), jnp.int32),
        pltpu.SemaphoreType.DMA,
        pltpu.SemaphoreType.DMA,
    ],
)
def _sc_degree(dst_hbm, out_hbm, acc, zbuf, ones_v, didx, stage_sem, add_sem):
    c = lax.axis_index("c")
    s = lax.axis_index("s")

    def _fill(i, carry):
        zbuf[pl.ds(i * 16, 16)] = jnp.zeros((16,), jnp.float32)
        return carry

    lax.fori_loop(0, TROWS // 16, _fill, 0)
    for i in range(CW // 16):
        ones_v[pl.ds(i * 16, 16)] = jnp.ones((16,), jnp.float32)
    pltpu.sync_copy(zbuf, acc.at[pl.ds(s * TROWS, TROWS)])
    plsc.subcore_barrier()

    wid = c * NS + s

    def _group(k, carry):
        base = wid * CPW + k * GD
        pltpu.async_copy(dst_hbm.at[pl.ds(base, GD)], didx, stage_sem).wait()
        cps = []
        for j in range(GD):
            cps.append(pltpu.async_copy(ones_v, acc.at[didx.at[j]], add_sem, add=True))
        for cp in cps:
            cp.wait()
        return carry

    lax.fori_loop(0, CPW // GD, _group, 0)
    plsc.subcore_barrier()
    pltpu.sync_copy(acc.at[pl.ds(s * TROWS, TROWS)], zbuf)
    pltpu.sync_copy(zbuf, out_hbm.at[pl.ds(c * NP + s * TROWS, TROWS)])


@functools.partial(
    pl.kernel,
    out_type=jax.ShapeDtypeStruct((NC * NP, HH), jnp.float32),
    mesh=_mesh,
    compiler_params=pltpu.CompilerParams(use_tc_tiling_on_sc=False),
    scratch_types=[
        pltpu.VMEM_SHARED((NP, HH), jnp.float32),
        pltpu.VMEM((G, CW), jnp.int32),
        pltpu.VMEM((G, CW), jnp.int32),
        pltpu.VMEM((G, CW), jnp.int32),
        pltpu.VMEM((G, CW), jnp.int32),
        pltpu.VMEM((G, CW, HH), jnp.float32),
        pltpu.VMEM((G, CW, HH), jnp.float32),
        pltpu.SemaphoreType.DMA,
        pltpu.SemaphoreType.DMA,
        pltpu.SemaphoreType.DMA,
        pltpu.SemaphoreType.DMA,
        pltpu.SemaphoreType.DMA,
    ],
)
def _sc_layer(g2_hbm, src_hbm, dst_hbm, out_hbm, acc, sidx_a, didx_a, sidx_b,
              didx_b, rows_a, rows_b, stage_sem, gat_sem_a, gat_sem_b,
              add_sem_a, add_sem_b):
    c = lax.axis_index("c")
    s = lax.axis_index("s")
    off = c * NP

    pltpu.sync_copy(g2_hbm.at[pl.ds(off + s * TROWS, TROWS)],
                    acc.at[pl.ds(s * TROWS, TROWS)])
    plsc.subcore_barrier()

    sbase = c * NCHUNK + s * CPT
    dbase = s * CPT

    def _stage(k, sidx, didx):
        pltpu.async_copy(src_hbm.at[pl.ds(sbase + k * G, G)], sidx, stage_sem)
        pltpu.async_copy(dst_hbm.at[pl.ds(dbase + k * G, G)], didx, stage_sem)

    def _wait_stage(sidx, didx):
        pltpu.make_async_copy(src_hbm.at[pl.ds(sbase, G)], sidx,
                              stage_sem).wait()
        pltpu.make_async_copy(dst_hbm.at[pl.ds(dbase, G)], didx,
                              stage_sem).wait()

    def _gathers(sidx, rows, sem):
        for j in range(G):
            pltpu.async_copy(g2_hbm.at[sidx.at[j]], rows.at[j], sem)

    def _wait_gathers(sidx, rows, sem):
        for j in range(G):
            pltpu.make_async_copy(g2_hbm.at[sidx.at[j]], rows.at[j], sem).wait()

    def _adds(didx, rows, sem):
        for j in range(G):
            pltpu.async_copy(rows.at[j], acc.at[didx.at[j]], sem, add=True)

    def _wait_adds(didx, rows, sem):
        for j in range(G):
            pltpu.make_async_copy(rows.at[j], acc.at[didx.at[j]], sem).wait()

    def _turn(k_next, sidx, didx, rows, gsem, asem):
        _wait_gathers(sidx, rows, gsem)
        _adds(didx, rows, asem)
        _wait_adds(didx, rows, asem)
        _stage(k_next, sidx, didx)
        _wait_stage(sidx, didx)
        _gathers(sidx, rows, gsem)

    _stage(0, sidx_a, didx_a)
    _wait_stage(sidx_a, didx_a)
    _gathers(sidx_a, rows_a, gat_sem_a)
    _stage(1, sidx_b, didx_b)
    _wait_stage(sidx_b, didx_b)
    _gathers(sidx_b, rows_b, gat_sem_b)

    def _body(k2, carry):
        _turn(2 * k2 + 2, sidx_a, didx_a, rows_a, gat_sem_a, add_sem_a)
        _turn(2 * k2 + 3, sidx_b, didx_b, rows_b, gat_sem_b, add_sem_b)
        return carry

    lax.fori_loop(0, NG // 2 - 1, _body, 0)

    _wait_gathers(sidx_a, rows_a, gat_sem_a)
    _adds(didx_a, rows_a, add_sem_a)
    _wait_adds(didx_a, rows_a, add_sem_a)
    _wait_gathers(sidx_b, rows_b, gat_sem_b)
    _adds(didx_b, rows_b, add_sem_b)
    _wait_adds(didx_b, rows_b, add_sem_b)

    plsc.subcore_barrier()
    pltpu.sync_copy(acc.at[pl.ds(s * TROWS, TROWS)],
                    out_hbm.at[pl.ds(off + s * TROWS, TROWS)])


RB = 6272
NB = NP // RB

_full = lambda *shape: pl.BlockSpec(shape, lambda b: tuple(0 for _ in shape))
_rows = lambda *shape: pl.BlockSpec(shape, lambda b: (b,) + tuple(0 for _ in shape[1:]))
_half = pl.BlockSpec((NC, RB, HH), lambda b: (0, b, 0))


def _tc_front_body(x_ref, w_in_ref, b_in_ref, dv_ref, wg_ref, g2_ref):
    h = jnp.maximum(
        jnp.dot(x_ref[...], w_in_ref[...], preferred_element_type=jnp.float32)
        + b_in_ref[...][None, :], 0.0)
    hw = jnp.dot(h, wg_ref[...], preferred_element_type=jnp.float32)
    g = hw * dv_ref[...]
    g2_ref[0] = g[:, :HH]
    g2_ref[1] = g[:, HH:]


_tc_front = pl.pallas_call(
    _tc_front_body,
    grid=(NB,),
    in_specs=[_rows(RB, 8), _full(8, H), _full(H), _rows(RB, H), _full(H, H)],
    out_specs=_half,
    out_shape=jax.ShapeDtypeStruct((NC, NP, HH), jnp.float32),
)


def _pre_bn(acc2_ref, dv_ref, bg_ref):
    accf = jnp.concatenate([acc2_ref[0], acc2_ref[1]], axis=1)
    return accf * dv_ref[...] + bg_ref[...][None, :]


def _tc_stats_body(acc2_ref, dv_ref, bg_ref, stats_ref):
    b = pl.program_id(0)
    pre = _pre_bn(acc2_ref, dv_ref, bg_ref)
    row = lax.broadcasted_iota(jnp.int32, (RB, H), 0) + b * RB
    prem = jnp.where(row < N, pre, 0.0)

    @pl.when(b == 0)
    def _():
        stats_ref[...] = jnp.zeros((8, H), jnp.float32)

    stats_ref[0, :] += jnp.sum(prem, axis=0)
    stats_ref[1, :] += jnp.sum(prem * prem, axis=0)


_tc_stats = pl.pallas_call(
    _tc_stats_body,
    grid=(NB,),
    in_specs=[_half, _rows(RB, H), _full(H)],
    out_specs=_full(8, H),
    out_shape=jax.ShapeDtypeStruct((8, H), jnp.float32),
)


def _bn_relu(acc2_ref, dv_ref, bg_ref, stats_ref, gamma_ref, beta_ref):
    pre = _pre_bn(acc2_ref, dv_ref, bg_ref)
    mu = stats_ref[0, :] / float(N)
    var = stats_ref[1, :] / float(N) - mu * mu
    h = (pre - mu[None, :]) * lax.rsqrt(var + EPS)[None, :] * gamma_ref[...][None, :] \
        + beta_ref[...][None, :]
    return jnp.maximum(h, 0.0)


def _tc_apply_body(acc2_ref, dv_ref, bg_ref, stats_ref, gamma_ref, beta_ref,
                   wg_ref, g2_ref):
    h = _bn_relu(acc2_ref, dv_ref, bg_ref, stats_ref, gamma_ref, beta_ref)
    hw = jnp.dot(h, wg_ref[...], preferred_element_type=jnp.float32)
    g = hw * dv_ref[...]
    g2_ref[0] = g[:, :HH]
    g2_ref[1] = g[:, HH:]


_tc_apply = pl.pallas_call(
    _tc_apply_body,
    grid=(NB,),
    in_specs=[_half, _rows(RB, H), _full(H), _full(8, H), _full(H), _full(H),
              _full(H, H)],
    out_specs=_half,
    out_shape=jax.ShapeDtypeStruct((NC, NP, HH), jnp.float32),
)


def _tc_head_body(acc2_ref, dv_ref, bg_ref, stats_ref, gamma_ref, beta_ref,
                  wc1_ref, bc1_ref, wc2_ref, bc2_ref, out_ref):
    h = _bn_relu(acc2_ref, dv_ref, bg_ref, stats_ref, gamma_ref, beta_ref)
    t = jnp.maximum(
        jnp.dot(h, wc1_ref[...], preferred_element_type=jnp.float32)
        + bc1_ref[...][None, :], 0.0)
    out_ref[...] = jnp.dot(t, wc2_ref[...], preferred_element_type=jnp.float32) \
        + bc2_ref[...][None, :]


_tc_head = pl.pallas_call(
    _tc_head_body,
    grid=(NB,),
    in_specs=[_half, _rows(RB, H), _full(H), _full(8, H), _full(H), _full(H),
              _full(H, HH), _full(HH), _full(HH, 1), _full(1)],
    out_specs=_rows(RB, 1),
    out_shape=jax.ShapeDtypeStruct((NP, 1), jnp.float32),
)


def kernel(x, edge_index, W_in, b_in, Wg0, bg0, gamma0, beta0, Wg1, bg1, gamma1,
           beta1, Wg2, bg2, gamma2, beta2, Wc1, bc1, Wc2, bc2):
    pad = jnp.full((EP - E,), NP - 1, jnp.int32)
    src2 = jnp.concatenate([edge_index[0], pad]).reshape(NCHUNK, CW)
    dst2 = jnp.concatenate([edge_index[1], pad]).reshape(NCHUNK, CW)
    src_sh = jnp.concatenate([src2, src2 + NP])
    x8 = jnp.pad(x, ((0, NP - N), (0, 3)))
    w8 = jnp.pad(W_in, ((0, 3), (0, 0)))

    deg_flat = _sc_degree(dst2)
    deg = deg_flat[:N] + deg_flat[NP:NP + N] + 1.0
    dinv64 = jnp.pad(jnp.broadcast_to(lax.rsqrt(deg)[:, None], (N, H)),
                     ((0, NP - N), (0, 0)))

    g2 = _tc_front(x8, w8, b_in, dinv64, Wg0)

    for bg, gamma, beta, wg in ((bg0, gamma0, beta0, Wg1),
                                (bg1, gamma1, beta1, Wg2)):
        acc2 = _sc_layer(g2.reshape(NC * NP, HH), src_sh, dst2).reshape(NC, NP, HH)
        stats = _tc_stats(acc2, dinv64, bg)
        g2 = _tc_apply(acc2, dinv64, bg, stats, gamma, beta, wg)

    acc2 = _sc_layer(g2.reshape(NC * NP, HH), src_sh, dst2).reshape(NC, NP, HH)
    stats = _tc_stats(acc2, dinv64, bg2)
    out = _tc_head(acc2, dinv64, bg2, stats, gamma2, beta2, Wc1, bc1, Wc2, bc2)
    return out[:N, 0]


# speedup vs baseline: 17.4837x; 1.0022x over previous
import functools

import jax
import jax.numpy as jnp
from jax import lax
from jax.experimental import pallas as pl
from jax.experimental.pallas import tpu as pltpu
from jax.experimental.pallas import tpu_sc as plsc

N = 50000
H = 64
HH = 32
E = 800000
EPS = 1e-5

NC = 2
NS = 16
NP = 50176
TROWS = NP // NS
CW = 128
EP = 819200
NCHUNK = EP // CW
CPT = NCHUNK // NS
G = 2
NG = CPT // G
CPW = NCHUNK // (NC * NS)
GD = 8

_mesh = plsc.VectorSubcoreMesh(core_axis_name="c", subcore_axis_name="s")


@functools.partial(
    pl.kernel,
    out_type=jax.ShapeDtypeStruct((NC * NP,), jnp.float32),
    mesh=_mesh,
    compiler_params=pltpu.CompilerParams(use_tc_tiling_on_sc=False),
    scratch_types=[
        pltpu.VMEM_SHARED((NP,), jnp.float32),
        pltpu.VMEM((TROWS,), jnp.float32),
        pltpu.VMEM((CW,), jnp.float32),
        pltpu.VMEM((GD, CW), jnp.int32),
        pltpu.SemaphoreType.DMA,
        pltpu.SemaphoreType.DMA,
    ],
)
def _sc_degree(dst_hbm, out_hbm, acc, zbuf, ones_v, didx, stage_sem, add_sem):
    c = lax.axis_index("c")
    s = lax.axis_index("s")

    def _fill(i, carry):
        zbuf[pl.ds(i * 16, 16)] = jnp.zeros((16,), jnp.float32)
        return carry

    lax.fori_loop(0, TROWS // 16, _fill, 0)
    for i in range(CW // 16):
        ones_v[pl.ds(i * 16, 16)] = jnp.ones((16,), jnp.float32)
    pltpu.sync_copy(zbuf, acc.at[pl.ds(s * TROWS, TROWS)])
    plsc.subcore_barrier()

    wid = c * NS + s

    def _group(k, carry):
        base = wid * CPW + k * GD
        pltpu.async_copy(dst_hbm.at[pl.ds(base, GD)], didx, stage_sem).wait()
        cps = []
        for j in range(GD):
            cps.append(pltpu.async_copy(ones_v, acc.at[didx.at[j]], add_sem, add=True))
        for cp in cps:
            cp.wait()
        return carry

    lax.fori_loop(0, CPW // GD, _group, 0)
    plsc.subcore_barrier()
    pltpu.sync_copy(acc.at[pl.ds(s * TROWS, TROWS)], zbuf)
    pltpu.sync_copy(zbuf, out_hbm.at[pl.ds(c * NP + s * TROWS, TROWS)])


@functools.partial(
    pl.kernel,
    out_type=jax.ShapeDtypeStruct((NC * NP, HH), jnp.float32),
    mesh=_mesh,
    compiler_params=pltpu.CompilerParams(use_tc_tiling_on_sc=False),
    scratch_types=[
        pltpu.VMEM_SHARED((NP, HH), jnp.float32),
        pltpu.VMEM((G, CW), jnp.int32),
        pltpu.VMEM((G, CW), jnp.int32),
        pltpu.VMEM((G, CW), jnp.int32),
        pltpu.VMEM((G, CW), jnp.int32),
        pltpu.VMEM((G, CW, HH), jnp.float32),
        pltpu.VMEM((G, CW, HH), jnp.float32),
        pltpu.SemaphoreType.DMA,
        pltpu.SemaphoreType.DMA,
        pltpu.SemaphoreType.DMA,
        pltpu.SemaphoreType.DMA,
        pltpu.SemaphoreType.DMA,
    ],
)
def _sc_layer(g2_hbm, src_hbm, dst_hbm, out_hbm, acc, sidx_a, didx_a, sidx_b,
              didx_b, rows_a, rows_b, stage_sem, gat_sem_a, gat_sem_b,
              add_sem_a, add_sem_b):
    c = lax.axis_index("c")
    s = lax.axis_index("s")
    off = c * NP

    pltpu.sync_copy(g2_hbm.at[pl.ds(off + s * TROWS, TROWS)],
                    acc.at[pl.ds(s * TROWS, TROWS)])
    plsc.subcore_barrier()

    sbase = c * NCHUNK + s * CPT
    dbase = s * CPT

    def _stage(k, sidx, didx):
        pltpu.async_copy(src_hbm.at[pl.ds(sbase + k * G, G)], sidx, stage_sem)
        pltpu.async_copy(dst_hbm.at[pl.ds(dbase + k * G, G)], didx, stage_sem)

    def _wait_stage(sidx, didx):
        pltpu.make_async_copy(src_hbm.at[pl.ds(sbase, G)], sidx,
                              stage_sem).wait()
        pltpu.make_async_copy(dst_hbm.at[pl.ds(dbase, G)], didx,
                              stage_sem).wait()

    def _gathers(sidx, rows, sem):
        for j in range(G):
            pltpu.async_copy(g2_hbm.at[sidx.at[j]], rows.at[j], sem)

    def _wait_gathers(sidx, rows, sem):
        for j in range(G):
            pltpu.make_async_copy(g2_hbm.at[sidx.at[j]], rows.at[j], sem).wait()

    def _adds(didx, rows, sem):
        for j in range(G):
            pltpu.async_copy(rows.at[j], acc.at[didx.at[j]], sem, add=True)

    def _wait_adds(didx, rows, sem):
        for j in range(G):
            pltpu.make_async_copy(rows.at[j], acc.at[didx.at[j]], sem).wait()

    def _turn(k_next, sidx, didx, rows, gsem, asem):
        _wait_gathers(sidx, rows, gsem)
        _adds(didx, rows, asem)
        _wait_adds(didx, rows, asem)
        _stage(k_next, sidx, didx)
        _wait_stage(sidx, didx)
        _gathers(sidx, rows, gsem)

    _stage(0, sidx_a, didx_a)
    _wait_stage(sidx_a, didx_a)
    _gathers(sidx_a, rows_a, gat_sem_a)
    _stage(1, sidx_b, didx_b)
    _wait_stage(sidx_b, didx_b)
    _gathers(sidx_b, rows_b, gat_sem_b)

    def _body(k2, carry):
        _turn(2 * k2 + 2, sidx_a, didx_a, rows_a, gat_sem_a, add_sem_a)
        _turn(2 * k2 + 3, sidx_b, didx_b, rows_b, gat_sem_b, add_sem_b)
        return carry

    lax.fori_loop(0, NG // 2 - 1, _body, 0)

    _wait_gathers(sidx_a, rows_a, gat_sem_a)
    _adds(didx_a, rows_a, add_sem_a)
    _wait_adds(didx_a, rows_a, add_sem_a)
    _wait_gathers(sidx_b, rows_b, gat_sem_b)
    _adds(didx_b, rows_b, add_sem_b)
    _wait_adds(didx_b, rows_b, add_sem_b)

    plsc.subcore_barrier()
    pltpu.sync_copy(acc.at[pl.ds(s * TROWS, TROWS)],
                    out_hbm.at[pl.ds(off + s * TROWS, TROWS)])


RB = 6272
NB = NP // RB

_full = lambda *shape: pl.BlockSpec(shape, lambda b: tuple(0 for _ in shape))
_whole = lambda *shape: pl.BlockSpec(shape, lambda: tuple(0 for _ in shape))
_rows = lambda *shape: pl.BlockSpec(shape, lambda b: (b,) + tuple(0 for _ in shape[1:]))
_half = pl.BlockSpec((NC, RB, HH), lambda b: (0, b, 0))


def _tc_front_body(x_ref, w_in_ref, b_in_ref, dv_ref, wg_ref, g2_ref):
    h = jnp.maximum(
        jnp.dot(x_ref[...], w_in_ref[...], preferred_element_type=jnp.float32)
        + b_in_ref[...][None, :], 0.0)
    hw = jnp.dot(h, wg_ref[...], preferred_element_type=jnp.float32)
    g = hw * dv_ref[...]
    g2_ref[0] = g[:, :HH]
    g2_ref[1] = g[:, HH:]


_tc_front = pl.pallas_call(
    _tc_front_body,
    grid=(NB,),
    in_specs=[_rows(RB, 8), _full(8, H), _full(H), _rows(RB, H), _full(H, H)],
    out_specs=_half,
    out_shape=jax.ShapeDtypeStruct((NC, NP, HH), jnp.float32),
)


def _tc_stats_body(acc2_ref, dv_ref, st_ref):
    ad = jnp.concatenate([acc2_ref[0], acc2_ref[1]], axis=1) * dv_ref[...]
    part = jnp.concatenate(
        [jnp.sum(ad, axis=0)[None, :], jnp.sum(ad * ad, axis=0)[None, :],
         jnp.zeros((6, H), jnp.float32)], axis=0)
    b = pl.program_id(0)

    @pl.when(b == 0)
    def _():
        st_ref[...] = part

    @pl.when(b != 0)
    def _():
        st_ref[...] = st_ref[...] + part


_tc_stats = pl.pallas_call(
    _tc_stats_body,
    grid=(NB,),
    in_specs=[_half, _rows(RB, H)],
    out_specs=_full(8, H),
    out_shape=jax.ShapeDtypeStruct((8, H), jnp.float32),
)


def _bn_relu_blk(acc2_ref, dv_ref, st_ref, gamma_ref, beta_ref):
    ad = jnp.concatenate([acc2_ref[0], acc2_ref[1]], axis=1) * dv_ref[...]
    mu = st_ref[0] / float(N)
    var = st_ref[1] / float(N) - mu * mu
    h = (ad - mu[None, :]) * lax.rsqrt(var + EPS)[None, :] * gamma_ref[...][None, :] \
        + beta_ref[...][None, :]
    return jnp.maximum(h, 0.0)


def _tc_mid_body(acc2_ref, dv_ref, st_ref, gamma_ref, beta_ref, wg_ref, g2_ref):
    h = _bn_relu_blk(acc2_ref, dv_ref, st_ref, gamma_ref, beta_ref)
    hw = jnp.dot(h, wg_ref[...], preferred_element_type=jnp.float32)
    g = hw * dv_ref[...]
    g2_ref[0] = g[:, :HH]
    g2_ref[1] = g[:, HH:]


_tc_mid = pl.pallas_call(
    _tc_mid_body,
    grid=(NB,),
    in_specs=[_half, _rows(RB, H), _full(8, H), _full(H), _full(H),
              _full(H, H)],
    out_specs=_half,
    out_shape=jax.ShapeDtypeStruct((NC, NP, HH), jnp.float32),
)


def _tc_head_body(acc2_ref, dv_ref, st_ref, gamma_ref, beta_ref, wc1_ref,
                  bc1_ref, wc2_ref, bc2_ref, out_ref):
    h = _bn_relu_blk(acc2_ref, dv_ref, st_ref, gamma_ref, beta_ref)
    t = jnp.maximum(
        jnp.dot(h, wc1_ref[...], preferred_element_type=jnp.float32)
        + bc1_ref[...][None, :], 0.0)
    out_ref[...] = jnp.dot(t, wc2_ref[...], preferred_element_type=jnp.float32) \
        + bc2_ref[...][None, :]


_tc_head = pl.pallas_call(
    _tc_head_body,
    grid=(NB,),
    in_specs=[_half, _rows(RB, H), _full(8, H), _full(H), _full(H),
              _full(H, HH), _full(HH), _full(HH, 1), _full(1)],
    out_specs=_rows(RB, 1),
    out_shape=jax.ShapeDtypeStruct((NP, 1), jnp.float32),
)


def kernel(x, edge_index, W_in, b_in, Wg0, bg0, gamma0, beta0, Wg1, bg1, gamma1,
           beta1, Wg2, bg2, gamma2, beta2, Wc1, bc1, Wc2, bc2):
    pad = jnp.full((EP - E,), NP - 1, jnp.int32)
    src2 = jnp.concatenate([edge_index[0], pad]).reshape(NCHUNK, CW)
    dst2 = jnp.concatenate([edge_index[1], pad]).reshape(NCHUNK, CW)
    src_sh = jnp.concatenate([src2, src2 + NP])
    x8 = jnp.pad(x, ((0, NP - N), (0, 3)))
    w8 = jnp.pad(W_in, ((0, 3), (0, 0)))

    deg_flat = _sc_degree(dst2)
    deg = deg_flat[:N] + deg_flat[NP:NP + N] + 1.0
    dinv64 = jnp.pad(jnp.broadcast_to(lax.rsqrt(deg)[:, None], (N, H)),
                     ((0, NP - N), (0, 0)))

    g2 = _tc_front(x8, w8, b_in, dinv64, Wg0)

    for gamma, beta, wg in ((gamma0, beta0, Wg1), (gamma1, beta1, Wg2)):
        acc2 = _sc_layer(g2.reshape(NC * NP, HH), src_sh, dst2).reshape(NC, NP, HH)
        st = _tc_stats(acc2, dinv64)
        g2 = _tc_mid(acc2, dinv64, st, gamma, beta, wg)

    acc2 = _sc_layer(g2.reshape(NC * NP, HH), src_sh, dst2).reshape(NC, NP, HH)
    st = _tc_stats(acc2, dinv64)
    out = _tc_head(acc2, dinv64, st, gamma2, beta2, Wc1, bc1, Wc2, bc2)
    return out[:N, 0]
